# Initial kernel scaffold; baseline (speedup 1.0000x reference)
import jax
import jax.numpy as jnp
from jax.experimental import pallas as pl


def kernel(target_h, peer_h, peer_mask, WQ, bQ, WK, bK, WV, bV, W1, b1, W2, b2, gamma, beta):
    raise NotImplementedError("write your pallas kernel here")


# speedup vs baseline: 14.1362x; 14.1362x over previous
import math
import jax
import jax.numpy as jnp
from jax.experimental import pallas as pl
from jax.experimental.pallas import tpu as pltpu

_LAGS = (1, 5, 10, 21, 30)
_K = 8


def _fused_kernel(target_ref, peer_ref, pm_ref, wq_ref, bq_ref, wk_ref, bk_ref,
                  wv_ref, bv_ref, w1_ref, b1_ref, w2_ref, b2_ref, g_ref, bt_ref,
                  out_ref):
    f32 = jnp.float32
    _, T, H = target_ref.shape
    N = peer_ref.shape[1]
    L = len(_LAGS)
    neg = f32(-jnp.inf)
    scale = f32(1.0 / math.sqrt(H))

    bf16 = jnp.bfloat16
    x = target_ref[0]
    Q = jnp.dot(x, wq_ref[...], preferred_element_type=f32) + bq_ref[...]
    ph = peer_ref[0].reshape(N * T, H)
    Kp = (jnp.dot(ph, wk_ref[...], preferred_element_type=f32)
          + bk_ref[...]).reshape(N, T, H)
    Vp = (jnp.dot(ph, wv_ref[...], preferred_element_type=f32)
          + bv_ref[...]).reshape(N, T, H)
    pm = pm_ref[0]

    Qb = Q.astype(bf16).astype(f32)
    Kb = Kp.astype(bf16).astype(f32)

    rows = []
    for lag in _LAGS:
        s = jnp.sum(Qb[None, lag:, :] * Kb[:, :T - lag, :], axis=-1) * scale
        s = jnp.where(pm > 0, s, neg)
        s = jnp.concatenate([jnp.full((N, lag), neg, f32), s], axis=1)
        rows.append(s)
    lg = jnp.concatenate(rows, axis=0)

    work = lg
    m1 = None
    tau = None
    for i in range(_K):
        m = jnp.max(work, axis=0, keepdims=True)
        if i == 0:
            m1 = m
        tau = m
        work = jnp.where(work == m, neg, work)

    m1s = jnp.where(m1 == neg, f32(0), m1)
    e = jnp.where(lg == neg, f32(0), jnp.exp(lg - m1s))
    w = jnp.where(lg >= tau, e, f32(0))
    denom = jnp.sum(w, axis=0, keepdims=True)
    w = jnp.where(denom > 0, w / denom, f32(0))

    wt = jnp.transpose(w)
    cs = jnp.zeros((T, H), f32)
    for i, lag in enumerate(_LAGS):
        acc = jnp.zeros((T - lag, H), f32)
        for n in range(N):
            col = i * N + n
            acc = acc + wt[lag:, col:col + 1] * Vp[n, :T - lag, :]
        cs = cs + jnp.concatenate([jnp.zeros((lag, H), f32), acc], axis=0)

    h1 = jnp.dot(cs.astype(bf16), w1_ref[...],
                 preferred_element_type=f32) + b1_ref[...]
    h1 = jnp.where(h1 > 0, h1, jnp.exp(jnp.minimum(h1, f32(0))) - f32(1))
    ffn = jnp.dot(h1.astype(bf16), w2_ref[...],
                  preferred_element_type=f32) + b2_ref[...]
    y = cs + ffn
    mu = jnp.mean(y, axis=-1, keepdims=True)
    var = jnp.mean((y - mu) ** 2, axis=-1, keepdims=True)
    out_ref[0] = g_ref[...] * (y - mu) / jnp.sqrt(var + f32(1e-5)) + bt_ref[...]


def kernel(target_h, peer_h, peer_mask, WQ, bQ, WK, bK, WV, bV, W1, b1, W2, b2,
           gamma, beta):
    B, N, T, H = peer_h.shape
    f32 = jnp.float32
    bf16 = jnp.bfloat16
    pm = peer_mask.astype(f32).reshape(B, N, 1)
    target_b = target_h.astype(bf16)
    peer_b = peer_h.astype(bf16)
    wq, wk, wv, w1, w2 = (jnp.transpose(W).astype(bf16)
                          for W in (WQ, WK, WV, W1, W2))
    bq, bk, bv, b1r, b2r, g, bt = (v.reshape(1, H).astype(f32)
                                   for v in (bQ, bK, bV, b1, b2, gamma, beta))

    full = lambda b: (0, 0)
    grid_spec = pl.GridSpec(
        grid=(B,),
        in_specs=[
            pl.BlockSpec((1, T, H), lambda b: (b, 0, 0)),
            pl.BlockSpec((1, N, T, H), lambda b: (b, 0, 0, 0)),
            pl.BlockSpec((1, N, 1), lambda b: (b, 0, 0)),
            pl.BlockSpec((H, H), full),
            pl.BlockSpec((1, H), full),
            pl.BlockSpec((H, H), full),
            pl.BlockSpec((1, H), full),
            pl.BlockSpec((H, H), full),
            pl.BlockSpec((1, H), full),
            pl.BlockSpec((H, H), full),
            pl.BlockSpec((1, H), full),
            pl.BlockSpec((H, H), full),
            pl.BlockSpec((1, H), full),
            pl.BlockSpec((1, H), full),
            pl.BlockSpec((1, H), full),
        ],
        out_specs=pl.BlockSpec((1, T, H), lambda b: (b, 0, 0)),
    )
    return pl.pallas_call(
        _fused_kernel,
        grid_spec=grid_spec,
        out_shape=jax.ShapeDtypeStruct((B, T, H), f32),
    )(target_b, peer_b, pm, wq, bq, wk, bk, wv, bv, w1, b1r, w2, b2r, g, bt)


# speedup vs baseline: 17.9611x; 1.2706x over previous
import math
import jax
import jax.numpy as jnp
from jax.experimental import pallas as pl
from jax.experimental.pallas import tpu as pltpu

_LAGS = (1, 5, 10, 21, 30)
_K = 8


def _fused_kernel(xT_ref, peer_ref, pm_ref, wq_ref, bq_ref, wk_ref, bk_ref,
                  wv_ref, bv_ref, w1_ref, b1_ref, w2_ref, b2_ref, g_ref,
                  bt_ref, out_ref):
    f32 = jnp.float32
    bf16 = jnp.bfloat16
    _, H, T = xT_ref.shape
    N = peer_ref.shape[1]
    L = len(_LAGS)
    neg = f32(-jnp.inf)
    scale = f32(1.0 / math.sqrt(H))

    bkT = jnp.broadcast_to(bk_ref[...], (H, T))
    bvT = jnp.broadcast_to(bv_ref[...], (H, T))

    xT = xT_ref[0]
    QT = jnp.dot(wq_ref[...], xT, preferred_element_type=f32) \
        + jnp.broadcast_to(bq_ref[...], (H, T))
    Qb = QT.astype(bf16).astype(f32)
    ph = peer_ref[0]

    Kb = []
    Vp = []
    for n in range(N):
        kn = jnp.dot(wk_ref[...], ph[n], preferred_element_type=f32) + bkT
        Kb.append(kn.astype(bf16).astype(f32))
        Vp.append(jnp.dot(wv_ref[...], ph[n], preferred_element_type=f32)
                  + bvT)
    pm = pm_ref[0]

    zrow = jnp.zeros((H, max(_LAGS)), f32)
    rows = []
    for lag in _LAGS:
        Qs = jnp.concatenate([Qb[:, lag:], zrow[:, :lag]], axis=1)
        sl = jnp.concatenate(
            [jnp.sum(Qs * Kb[n], axis=0, keepdims=True) for n in range(N)],
            axis=0) * scale
        sl = jnp.where(pm > 0, sl, neg)
        sl = jnp.concatenate([jnp.full((N, lag), neg, f32), sl[:, :T - lag]],
                             axis=1)
        rows.append(sl)
    lg = jnp.concatenate(rows, axis=0)

    work = lg
    m1 = None
    tau = None
    for i in range(_K):
        m = jnp.max(work, axis=0, keepdims=True)
        if i == 0:
            m1 = m
        tau = m
        work = jnp.where(work == m, neg, work)

    m1s = jnp.where(m1 == neg, f32(0), m1)
    e = jnp.where(lg == neg, f32(0), jnp.exp(lg - m1s))
    w = jnp.where(lg >= tau, e, f32(0))
    denom = jnp.sum(w, axis=0, keepdims=True)
    w = jnp.where(denom > 0, w / denom, f32(0))

    zn = jnp.zeros((N, max(_LAGS)), f32)
    zh = jnp.zeros((H, max(_LAGS)), f32)
    csT = jnp.zeros((H, T), f32)
    for i, lag in enumerate(_LAGS):
        wblk = w[i * N:(i + 1) * N, :]
        wsh = jnp.concatenate([wblk[:, lag:], zn[:, :lag]], axis=1)
        acc = wsh[0:1, :] * Vp[0]
        for n in range(1, N):
            acc = acc + wsh[n:n + 1, :] * Vp[n]
        csT = csT + jnp.concatenate([zh[:, :lag], acc[:, :T - lag]], axis=1)

    h1 = jnp.dot(w1_ref[...], csT.astype(bf16), preferred_element_type=f32) \
        + jnp.broadcast_to(b1_ref[...], (H, T))
    h1 = jnp.where(h1 > 0, h1, jnp.exp(jnp.minimum(h1, f32(0))) - f32(1))
    ffn = jnp.dot(w2_ref[...], h1.astype(bf16), preferred_element_type=f32) \
        + jnp.broadcast_to(b2_ref[...], (H, T))
    y = csT + ffn
    mu = jnp.mean(y, axis=0, keepdims=True)
    var = jnp.mean((y - mu) ** 2, axis=0, keepdims=True)
    yn = g_ref[...] * (y - mu) / jnp.sqrt(var + f32(1e-5)) + bt_ref[...]
    out_ref[0] = jnp.transpose(yn)


def kernel(target_h, peer_h, peer_mask, WQ, bQ, WK, bK, WV, bV, W1, b1, W2, b2,
           gamma, beta):
    B, N, T, H = peer_h.shape
    f32 = jnp.float32
    bf16 = jnp.bfloat16
    pm = peer_mask.astype(f32).reshape(B, N, 1)
    xT = jnp.swapaxes(target_h, 1, 2).astype(bf16)
    phT = jnp.swapaxes(peer_h, 2, 3).astype(bf16)
    wq, wk, wv, w1, w2 = (W.astype(bf16) for W in (WQ, WK, WV, W1, W2))
    bq, bk, bv, b1r, b2r, g, bt = (v.reshape(H, 1).astype(f32)
                                   for v in (bQ, bK, bV, b1, b2, gamma, beta))

    full = lambda b: (0, 0)
    grid_spec = pl.GridSpec(
        grid=(B,),
        in_specs=[
            pl.BlockSpec((1, H, T), lambda b: (b, 0, 0)),
            pl.BlockSpec((1, N, H, T), lambda b: (b, 0, 0, 0)),
            pl.BlockSpec((1, N, 1), lambda b: (b, 0, 0)),
            pl.BlockSpec((H, H), full),
            pl.BlockSpec((H, 1), full),
            pl.BlockSpec((H, H), full),
            pl.BlockSpec((H, 1), full),
            pl.BlockSpec((H, H), full),
            pl.BlockSpec((H, 1), full),
            pl.BlockSpec((H, H), full),
            pl.BlockSpec((H, 1), full),
            pl.BlockSpec((H, H), full),
            pl.BlockSpec((H, 1), full),
            pl.BlockSpec((H, 1), full),
            pl.BlockSpec((H, 1), full),
        ],
        out_specs=pl.BlockSpec((1, T, H), lambda b: (b, 0, 0)),
    )
    return pl.pallas_call(
        _fused_kernel,
        grid_spec=grid_spec,
        out_shape=jax.ShapeDtypeStruct((B, T, H), f32),
    )(xT, phT, pm, wq, bq, wk, bk, wv, bv, w1, b1r, w2, b2r, g, bt)


# speedup vs baseline: 23.3120x; 1.2979x over previous
import math
import jax
import jax.numpy as jnp
from jax.experimental import pallas as pl
from jax.experimental.pallas import tpu as pltpu

_LAGS = (1, 5, 10, 21, 30)
_K = 8


def _fused_kernel(xT_ref, peer_ref, pm_ref, wq_ref, bq_ref, wk_ref, bk_ref,
                  wv_ref, bv_ref, w1_ref, b1_ref, w2_ref, b2_ref, g_ref,
                  bt_ref, out_ref):
    f32 = jnp.float32
    bf16 = jnp.bfloat16
    _, T, H = xT_ref.shape
    N = peer_ref.shape[1]
    L = len(_LAGS)
    neg = f32(-jnp.inf)
    scale = f32(1.0 / math.sqrt(H))

    bkT = jnp.broadcast_to(bk_ref[...], (H, T))
    bvT = jnp.broadcast_to(bv_ref[...], (H, T))
    dn_t = (((1,), (1,)), ((), ()))

    x = xT_ref[0].astype(bf16)
    QT = jax.lax.dot_general(wq_ref[...], x, dn_t,
                             preferred_element_type=f32) \
        + jnp.broadcast_to(bq_ref[...], (H, T))
    Qb = QT.astype(bf16).astype(f32)
    ph = peer_ref[0]

    Kb = []
    Vp = []
    for n in range(N):
        phb = ph[n].astype(bf16)
        kn = jax.lax.dot_general(wk_ref[...], phb, dn_t,
                                 preferred_element_type=f32) + bkT
        Kb.append(kn.astype(bf16).astype(f32))
        Vp.append(jax.lax.dot_general(wv_ref[...], phb, dn_t,
                                      preferred_element_type=f32) + bvT)
    pm = pm_ref[0]

    zrow = jnp.zeros((H, max(_LAGS)), f32)
    rows = []
    for lag in _LAGS:
        Qs = jnp.concatenate([Qb[:, lag:], zrow[:, :lag]], axis=1)
        sl = jnp.concatenate(
            [jnp.sum(Qs * Kb[n], axis=0, keepdims=True) for n in range(N)],
            axis=0) * scale
        sl = jnp.where(pm > 0, sl, neg)
        sl = jnp.concatenate([jnp.full((N, lag), neg, f32), sl[:, :T - lag]],
                             axis=1)
        rows.append(sl)
    lg = jnp.concatenate(rows, axis=0)

    work = lg
    m1 = None
    tau = None
    for i in range(_K):
        m = jnp.max(work, axis=0, keepdims=True)
        if i == 0:
            m1 = m
        tau = m
        work = jnp.where(work == m, neg, work)

    m1s = jnp.where(m1 == neg, f32(0), m1)
    e = jnp.where(lg == neg, f32(0), jnp.exp(lg - m1s))
    w = jnp.where(lg >= tau, e, f32(0))
    denom = jnp.sum(w, axis=0, keepdims=True)
    w = jnp.where(denom > 0, w / denom, f32(0))

    zn = jnp.zeros((N, max(_LAGS)), f32)
    zh = jnp.zeros((H, max(_LAGS)), f32)
    csT = jnp.zeros((H, T), f32)
    for i, lag in enumerate(_LAGS):
        wblk = w[i * N:(i + 1) * N, :]
        wsh = jnp.concatenate([wblk[:, lag:], zn[:, :lag]], axis=1)
        acc = wsh[0:1, :] * Vp[0]
        for n in range(1, N):
            acc = acc + wsh[n:n + 1, :] * Vp[n]
        csT = csT + jnp.concatenate([zh[:, :lag], acc[:, :T - lag]], axis=1)

    h1 = jnp.dot(w1_ref[...], csT.astype(bf16), preferred_element_type=f32) \
        + jnp.broadcast_to(b1_ref[...], (H, T))
    h1 = jnp.where(h1 > 0, h1, jnp.exp(jnp.minimum(h1, f32(0))) - f32(1))
    ffn = jnp.dot(w2_ref[...], h1.astype(bf16), preferred_element_type=f32) \
        + jnp.broadcast_to(b2_ref[...], (H, T))
    y = csT + ffn
    mu = jnp.mean(y, axis=0, keepdims=True)
    var = jnp.mean((y - mu) ** 2, axis=0, keepdims=True)
    yn = g_ref[...] * (y - mu) / jnp.sqrt(var + f32(1e-5)) + bt_ref[...]
    out_ref[0] = jnp.transpose(yn)


def kernel(target_h, peer_h, peer_mask, WQ, bQ, WK, bK, WV, bV, W1, b1, W2, b2,
           gamma, beta):
    B, N, T, H = peer_h.shape
    f32 = jnp.float32
    bf16 = jnp.bfloat16
    pm = peer_mask.astype(f32).reshape(B, N, 1)
    wq, wk, wv, w1, w2 = (W.astype(bf16) for W in (WQ, WK, WV, W1, W2))
    bq, bk, bv, b1r, b2r, g, bt = (v.reshape(H, 1).astype(f32)
                                   for v in (bQ, bK, bV, b1, b2, gamma, beta))

    full = lambda b: (0, 0)
    grid_spec = pl.GridSpec(
        grid=(B,),
        in_specs=[
            pl.BlockSpec((1, T, H), lambda b: (b, 0, 0)),
            pl.BlockSpec((1, N, T, H), lambda b: (b, 0, 0, 0)),
            pl.BlockSpec((1, N, 1), lambda b: (b, 0, 0)),
            pl.BlockSpec((H, H), full),
            pl.BlockSpec((H, 1), full),
            pl.BlockSpec((H, H), full),
            pl.BlockSpec((H, 1), full),
            pl.BlockSpec((H, H), full),
            pl.BlockSpec((H, 1), full),
            pl.BlockSpec((H, H), full),
            pl.BlockSpec((H, 1), full),
            pl.BlockSpec((H, H), full),
            pl.BlockSpec((H, 1), full),
            pl.BlockSpec((H, 1), full),
            pl.BlockSpec((H, 1), full),
        ],
        out_specs=pl.BlockSpec((1, T, H), lambda b: (b, 0, 0)),
    )
    return pl.pallas_call(
        _fused_kernel,
        grid_spec=grid_spec,
        out_shape=jax.ShapeDtypeStruct((B, T, H), f32),
    )(target_h, peer_h, pm, wq, bq, wk, bk, wv, bv, w1, b1r, w2, b2r, g, bt)


# speedup vs baseline: 35.8023x; 1.5358x over previous
import math
import jax
import jax.numpy as jnp
from jax.experimental import pallas as pl
from jax.experimental.pallas import tpu as pltpu

_LAGS = (1, 5, 10, 21, 30)
_K = 8


def _fused_kernel(xT_ref, peer_ref, pm_ref, wq_ref, wk_ref, wv_ref, w1_ref,
                  w2_ref, bias_ref, out_ref):
    f32 = jnp.float32
    bf16 = jnp.bfloat16
    _, T, H = xT_ref.shape
    N = peer_ref.shape[1]
    L = len(_LAGS)
    neg = f32(-jnp.inf)
    scale = f32(1.0 / math.sqrt(H))

    bkT = jnp.broadcast_to(bias_ref[:, 1:2], (H, T))
    bvT = jnp.broadcast_to(bias_ref[:, 2:3], (H, T))
    dn_t = (((1,), (1,)), ((), ()))

    x = xT_ref[0].astype(bf16)
    wq = wq_ref[...].astype(bf16)
    wk = wk_ref[...].astype(bf16)
    wv = wv_ref[...].astype(bf16)
    QT = jax.lax.dot_general(wq, x, dn_t, preferred_element_type=f32) \
        + jnp.broadcast_to(bias_ref[:, 0:1], (H, T))
    Qb = QT.astype(bf16).astype(f32)
    ph = peer_ref[0]

    Kb = []
    Vp = []
    for n in range(N):
        phb = ph[n].astype(bf16)
        kn = jax.lax.dot_general(wk, phb, dn_t,
                                 preferred_element_type=f32) + bkT
        Kb.append(kn.astype(bf16).astype(f32))
        Vp.append(jax.lax.dot_general(wv, phb, dn_t,
                                      preferred_element_type=f32) + bvT)
    pm = pm_ref[0]

    zrow = jnp.zeros((H, max(_LAGS)), f32)
    rows = []
    for lag in _LAGS:
        Qs = jnp.concatenate([Qb[:, lag:], zrow[:, :lag]], axis=1)
        sl = jnp.concatenate(
            [jnp.sum(Qs * Kb[n], axis=0, keepdims=True) for n in range(N)],
            axis=0) * scale
        sl = jnp.where(pm > 0, sl, neg)
        sl = jnp.concatenate([jnp.full((N, lag), neg, f32), sl[:, :T - lag]],
                             axis=1)
        rows.append(sl)
    lg = jnp.concatenate(rows, axis=0)

    work = lg
    m1 = None
    tau = None
    for i in range(_K):
        m = jnp.max(work, axis=0, keepdims=True)
        if i == 0:
            m1 = m
        tau = m
        work = jnp.where(work == m, neg, work)

    m1s = jnp.where(m1 == neg, f32(0), m1)
    e = jnp.where(lg == neg, f32(0), jnp.exp(lg - m1s))
    w = jnp.where(lg >= tau, e, f32(0))
    denom = jnp.sum(w, axis=0, keepdims=True)
    w = jnp.where(denom > 0, w / denom, f32(0))

    zn = jnp.zeros((N, max(_LAGS)), f32)
    zh = jnp.zeros((H, max(_LAGS)), f32)
    csT = jnp.zeros((H, T), f32)
    for i, lag in enumerate(_LAGS):
        wblk = w[i * N:(i + 1) * N, :]
        wsh = jnp.concatenate([wblk[:, lag:], zn[:, :lag]], axis=1)
        acc = wsh[0:1, :] * Vp[0]
        for n in range(1, N):
            acc = acc + wsh[n:n + 1, :] * Vp[n]
        csT = csT + jnp.concatenate([zh[:, :lag], acc[:, :T - lag]], axis=1)

    h1 = jnp.dot(w1_ref[...].astype(bf16), csT.astype(bf16),
                 preferred_element_type=f32) \
        + jnp.broadcast_to(bias_ref[:, 3:4], (H, T))
    h1 = jnp.where(h1 > 0, h1, jnp.exp(jnp.minimum(h1, f32(0))) - f32(1))
    ffn = jnp.dot(w2_ref[...].astype(bf16), h1.astype(bf16),
                  preferred_element_type=f32) \
        + jnp.broadcast_to(bias_ref[:, 4:5], (H, T))
    y = csT + ffn
    mu = jnp.mean(y, axis=0, keepdims=True)
    var = jnp.mean((y - mu) ** 2, axis=0, keepdims=True)
    yn = bias_ref[:, 5:6] * (y - mu) / jnp.sqrt(var + f32(1e-5)) \
        + bias_ref[:, 6:7]
    out_ref[0] = jnp.transpose(yn)


def kernel(target_h, peer_h, peer_mask, WQ, bQ, WK, bK, WV, bV, W1, b1, W2, b2,
           gamma, beta):
    B, N, T, H = peer_h.shape
    f32 = jnp.float32
    pm = peer_mask.astype(f32).reshape(B, N, 1)
    biases = jnp.stack([v.astype(f32) for v in
                        (bQ, bK, bV, b1, b2, gamma, beta)], axis=1)

    full = lambda b: (0, 0)
    grid_spec = pl.GridSpec(
        grid=(B,),
        in_specs=[
            pl.BlockSpec((1, T, H), lambda b: (b, 0, 0)),
            pl.BlockSpec((1, N, T, H), lambda b: (b, 0, 0, 0)),
            pl.BlockSpec((1, N, 1), lambda b: (b, 0, 0)),
            pl.BlockSpec((H, H), full),
            pl.BlockSpec((H, H), full),
            pl.BlockSpec((H, H), full),
            pl.BlockSpec((H, H), full),
            pl.BlockSpec((H, H), full),
            pl.BlockSpec((H, 7), full),
        ],
        out_specs=pl.BlockSpec((1, T, H), lambda b: (b, 0, 0)),
    )
    return pl.pallas_call(
        _fused_kernel,
        grid_spec=grid_spec,
        out_shape=jax.ShapeDtypeStruct((B, T, H), f32),
    )(target_h, peer_h, pm, WQ, WK, WV, W1, W2, biases)


# speedup vs baseline: 37.1092x; 1.0365x over previous
import math
import jax
import jax.numpy as jnp
from jax.experimental import pallas as pl
from jax.experimental.pallas import tpu as pltpu

_LAGS = (1, 5, 10, 21, 30)
_K = 8


def _fused_kernel(xT_ref, peer_ref, pm_ref, wq_ref, wk_ref, wv_ref, w1_ref,
                  w2_ref, bq_ref, bk_ref, bv_ref, b1_ref, b2_ref, g_ref,
                  bt_ref, out_ref):
    f32 = jnp.float32
    bf16 = jnp.bfloat16
    _, T, H = xT_ref.shape
    N = peer_ref.shape[1]
    L = len(_LAGS)
    neg = f32(-jnp.inf)
    scale = f32(1.0 / math.sqrt(H))

    bqC, bkC, bvC, b1C, b2C, gC, btC = (
        jnp.transpose(r[...]) for r in
        (bq_ref, bk_ref, bv_ref, b1_ref, b2_ref, g_ref, bt_ref))
    bkT = jnp.broadcast_to(bkC, (H, T))
    bvT = jnp.broadcast_to(bvC, (H, T))
    dn_t = (((1,), (1,)), ((), ()))

    x = xT_ref[0].astype(bf16)
    wq = wq_ref[...].astype(bf16)
    wk = wk_ref[...].astype(bf16)
    wv = wv_ref[...].astype(bf16)
    QT = jax.lax.dot_general(wq, x, dn_t, preferred_element_type=f32) \
        + jnp.broadcast_to(bqC, (H, T))
    Qb = QT.astype(bf16).astype(f32)
    ph = peer_ref[0]

    Kb = []
    Vp = []
    for n in range(N):
        phb = ph[n].astype(bf16)
        kn = jax.lax.dot_general(wk, phb, dn_t,
                                 preferred_element_type=f32) + bkT
        Kb.append(kn.astype(bf16).astype(f32))
        Vp.append(jax.lax.dot_general(wv, phb, dn_t,
                                      preferred_element_type=f32) + bvT)
    pm = pm_ref[0]

    zrow = jnp.zeros((H, max(_LAGS)), f32)
    rows = []
    for lag in _LAGS:
        Qs = jnp.concatenate([Qb[:, lag:], zrow[:, :lag]], axis=1)
        sl = jnp.concatenate(
            [jnp.sum(Qs * Kb[n], axis=0, keepdims=True) for n in range(N)],
            axis=0) * scale
        sl = jnp.where(pm > 0, sl, neg)
        sl = jnp.concatenate([jnp.full((N, lag), neg, f32), sl[:, :T - lag]],
                             axis=1)
        rows.append(sl)
    lg = jnp.concatenate(rows, axis=0)

    work = lg
    m1 = None
    tau = None
    for i in range(_K):
        m = jnp.max(work, axis=0, keepdims=True)
        if i == 0:
            m1 = m
        tau = m
        work = jnp.where(work == m, neg, work)

    m1s = jnp.where(m1 == neg, f32(0), m1)
    e = jnp.where(lg == neg, f32(0), jnp.exp(lg - m1s))
    w = jnp.where(lg >= tau, e, f32(0))
    denom = jnp.sum(w, axis=0, keepdims=True)
    w = jnp.where(denom > 0, w / denom, f32(0))

    zn = jnp.zeros((N, max(_LAGS)), f32)
    zh = jnp.zeros((H, max(_LAGS)), f32)
    csT = jnp.zeros((H, T), f32)
    for i, lag in enumerate(_LAGS):
        wblk = w[i * N:(i + 1) * N, :]
        wsh = jnp.concatenate([wblk[:, lag:], zn[:, :lag]], axis=1)
        acc = wsh[0:1, :] * Vp[0]
        for n in range(1, N):
            acc = acc + wsh[n:n + 1, :] * Vp[n]
        csT = csT + jnp.concatenate([zh[:, :lag], acc[:, :T - lag]], axis=1)

    h1 = jnp.dot(w1_ref[...].astype(bf16), csT.astype(bf16),
                 preferred_element_type=f32) \
        + jnp.broadcast_to(b1C, (H, T))
    h1 = jnp.where(h1 > 0, h1, jnp.exp(jnp.minimum(h1, f32(0))) - f32(1))
    ffn = jnp.dot(w2_ref[...].astype(bf16), h1.astype(bf16),
                  preferred_element_type=f32) \
        + jnp.broadcast_to(b2C, (H, T))
    y = csT + ffn
    mu = jnp.mean(y, axis=0, keepdims=True)
    var = jnp.mean((y - mu) ** 2, axis=0, keepdims=True)
    yn = gC * (y - mu) / jnp.sqrt(var + f32(1e-5)) + btC
    out_ref[0] = jnp.transpose(yn)


def kernel(target_h, peer_h, peer_mask, WQ, bQ, WK, bK, WV, bV, W1, b1, W2, b2,
           gamma, beta):
    B, N, T, H = peer_h.shape
    f32 = jnp.float32
    pm = peer_mask.astype(f32).reshape(B, N, 1)
    bq, bk, bv, b1r, b2r, g, bt = (v.reshape(1, H)
                                   for v in (bQ, bK, bV, b1, b2, gamma, beta))

    full = lambda b: (0, 0)
    grid_spec = pl.GridSpec(
        grid=(B,),
        in_specs=[
            pl.BlockSpec((1, T, H), lambda b: (b, 0, 0)),
            pl.BlockSpec((1, N, T, H), lambda b: (b, 0, 0, 0)),
            pl.BlockSpec((1, N, 1), lambda b: (b, 0, 0)),
            pl.BlockSpec((H, H), full),
            pl.BlockSpec((H, H), full),
            pl.BlockSpec((H, H), full),
            pl.BlockSpec((H, H), full),
            pl.BlockSpec((H, H), full),
            pl.BlockSpec((1, H), full),
            pl.BlockSpec((1, H), full),
            pl.BlockSpec((1, H), full),
            pl.BlockSpec((1, H), full),
            pl.BlockSpec((1, H), full),
            pl.BlockSpec((1, H), full),
            pl.BlockSpec((1, H), full),
        ],
        out_specs=pl.BlockSpec((1, T, H), lambda b: (b, 0, 0)),
    )
    return pl.pallas_call(
        _fused_kernel,
        grid_spec=grid_spec,
        out_shape=jax.ShapeDtypeStruct((B, T, H), f32),
    )(target_h, peer_h, pm, WQ, WK, WV, W1, W2,
      bq, bk, bv, b1r, b2r, g, bt)
